# Initial kernel scaffold; baseline (speedup 1.0000x reference)
import jax
import jax.numpy as jnp
from jax.experimental import pallas as pl


def kernel(hidden_states, gate_weight):
    raise NotImplementedError("write your pallas kernel here")


# speedup vs baseline: 1.6024x; 1.6024x over previous
import jax
import jax.numpy as jnp
from jax import lax
from jax.experimental import pallas as pl
from jax.experimental.pallas import tpu as pltpu

_E = 16
_CAP = 1024
_POS_INF_BITS = 0x7F800000


def _probs_body(h_ref, wt_ref, p_ref):
    x = h_ref[...]
    wt = wt_ref[...]
    logits = jnp.dot(x, wt, preferred_element_type=jnp.float32)
    logits = jnp.clip(logits, -10.0, 10.0)
    m = jnp.max(logits, axis=-1, keepdims=True)
    e = jnp.exp(logits - m)
    p_ref[...] = e / jnp.sum(e, axis=-1, keepdims=True)


def _select_body(p_ref, t_ref, i_ref, bits_ref):
    n = p_ref.shape[0]
    bits_ref[...] = lax.bitcast_convert_type(p_ref[...], jnp.int32)

    def val_body(_, carry):
        lo, hi = carry
        mid = lo + ((hi - lo + 1) >> 1)
        cnt = jnp.sum((bits_ref[...] >= mid).astype(jnp.int32), axis=0,
                      keepdims=True)
        ok = cnt >= _CAP
        return jnp.where(ok, mid, lo), jnp.where(ok, hi, mid - 1)

    lo0 = jnp.zeros((1, _E), jnp.int32)
    hi0 = jnp.full((1, _E), _POS_INF_BITS, jnp.int32)
    tbits, _ = lax.fori_loop(0, 31, val_body, (lo0, hi0))

    cgt = jnp.sum((bits_ref[...] > tbits).astype(jnp.int32), axis=0,
                  keepdims=True)
    need = _CAP - cgt

    idx = lax.broadcasted_iota(jnp.int32, (n, 1), 0)

    def idx_body(_, carry):
        lo, hi = carry
        mid = (lo + hi) >> 1
        cnt = jnp.sum(((bits_ref[...] == tbits) & (idx <= mid)).astype(
            jnp.int32), axis=0, keepdims=True)
        ok = cnt >= need
        return jnp.where(ok, lo, mid + 1), jnp.where(ok, mid, hi)

    ilo0 = jnp.zeros((1, _E), jnp.int32)
    ihi0 = jnp.full((1, _E), n - 1, jnp.int32)
    ilo, _ = lax.fori_loop(0, 13, idx_body, (ilo0, ihi0))

    t_ref[...] = tbits
    i_ref[...] = ilo


def _finalize_body(p_ref, t_ref, i_ref, w_ref, m_ref):
    blk = p_ref.shape[0]
    p = p_ref[...]
    bits = lax.bitcast_convert_type(p, jnp.int32)
    tbits = t_ref[...]
    icut = i_ref[...]
    base = pl.program_id(0) * blk
    idx = base + lax.broadcasted_iota(jnp.int32, (blk, 1), 0)
    mask = (bits > tbits) | ((bits == tbits) & (idx <= icut))
    maskf = mask.astype(jnp.float32)
    wun = maskf * p
    denom = jnp.sum(wun, axis=-1, keepdims=True) + 1e-10
    w_ref[...] = wun / denom
    m_ref[...] = maskf


def kernel(hidden_states, gate_weight):
    b, s, d = hidden_states.shape
    n = b * s
    h = hidden_states.reshape(n, d)
    wt = gate_weight.T

    tok_blk = 512
    probs = pl.pallas_call(
        _probs_body,
        grid=(n // tok_blk,),
        in_specs=[
            pl.BlockSpec((tok_blk, d), lambda i: (i, 0)),
            pl.BlockSpec((d, _E), lambda i: (0, 0)),
        ],
        out_specs=pl.BlockSpec((tok_blk, _E), lambda i: (i, 0)),
        out_shape=jax.ShapeDtypeStruct((n, _E), jnp.float32),
    )(h, wt)

    tbits, icut = pl.pallas_call(
        _select_body,
        in_specs=[pl.BlockSpec((n, _E), lambda: (0, 0))],
        out_specs=[
            pl.BlockSpec((1, _E), lambda: (0, 0)),
            pl.BlockSpec((1, _E), lambda: (0, 0)),
        ],
        out_shape=[
            jax.ShapeDtypeStruct((1, _E), jnp.int32),
            jax.ShapeDtypeStruct((1, _E), jnp.int32),
        ],
        scratch_shapes=[pltpu.VMEM((n, _E), jnp.int32)],
    )(probs)

    fin_blk = 1024
    w, m = pl.pallas_call(
        _finalize_body,
        grid=(n // fin_blk,),
        in_specs=[
            pl.BlockSpec((fin_blk, _E), lambda i: (i, 0)),
            pl.BlockSpec((1, _E), lambda i: (0, 0)),
            pl.BlockSpec((1, _E), lambda i: (0, 0)),
        ],
        out_specs=[
            pl.BlockSpec((fin_blk, _E), lambda i: (i, 0)),
            pl.BlockSpec((fin_blk, _E), lambda i: (i, 0)),
        ],
        out_shape=[
            jax.ShapeDtypeStruct((n, _E), jnp.float32),
            jax.ShapeDtypeStruct((n, _E), jnp.float32),
        ],
    )(probs, tbits, icut)

    return w.reshape(b, s, _E), m.reshape(b, s, _E)
